# baseline (speedup 1.0000x reference)
import functools

import jax
import jax.numpy as jnp
from jax import lax
from jax.experimental import pallas as pl
from jax.experimental.pallas import tpu as pltpu
from jax.experimental.pallas import tpu_sc as plsc

_LAM_U = 0.01
_D = 32
_PACK = 4
_W = _D * _PACK
_CHUNK = 128


@functools.cache
def _sc_gather_fn(B: int, NC: int, NS: int):
    NW = NC * NS
    b_per_w = B // NW
    n_chunks = b_per_w // _CHUNK
    mesh = plsc.VectorSubcoreMesh(core_axis_name="c", subcore_axis_name="s")

    @functools.partial(
        pl.kernel,
        mesh=mesh,
        compiler_params=pltpu.CompilerParams(use_tc_tiling_on_sc=True),
        out_type=jax.ShapeDtypeStruct((B, _W), jnp.float32),
        scratch_types=[
            pltpu.VMEM((8, _CHUNK), jnp.int32),
            pltpu.VMEM((b_per_w, _W), jnp.float32),
            pltpu.SemaphoreType.DMA,
        ],
    )
    def sc_gather(item_hbm, table_hbm, out_hbm, idx_v, rows_v, sem):
        wid = lax.axis_index("s") * NC + lax.axis_index("c")
        base = wid * b_per_w
        pltpu.sync_copy(item_hbm.at[wid], idx_v.at[pl.ds(0, n_chunks)])
        copies = []
        for j in range(n_chunks):
            copies.append(pltpu.async_copy(
                table_hbm.at[idx_v.at[j]],
                rows_v.at[pl.ds(j * _CHUNK, _CHUNK)],
                sem))
        for c in copies:
            c.wait()
        pltpu.sync_copy(rows_v, out_hbm.at[pl.ds(base, b_per_w)])

    return sc_gather


_TC_BLOCKS = 8


def _tc_loss_body(g_ref, oh_ref, y_ref, u4_ref, u_ref, o_ref):
    i = pl.program_id(0)
    g = g_ref[...]
    oh = oh_ref[...]
    u4 = u4_ref[...]
    dots = lax.dot_general(
        g, u4,
        dimension_numbers=(((1,), (1,)), ((), ())),
        precision=lax.Precision.HIGHEST)
    logits = jnp.sum(dots * oh, axis=1)
    x = logits.reshape(y_ref.shape)
    y = y_ref[...]
    bce = jnp.maximum(x, 0.0) - x * y + jnp.log1p(jnp.exp(-jnp.abs(x)))
    part = jnp.sum(bce)

    @pl.when(i == 0)
    def _init():
        u = u_ref[...]
        o_ref[0, 0] = _LAM_U * jnp.sqrt(jnp.sum(u * u))

    o_ref[0, 0] += part


def _tc_loss(gathered, oh, y2d, u4, u):
    B = gathered.shape[0]
    Bb = B // _TC_BLOCKS
    Rb = Bb // 128
    return pl.pallas_call(
        _tc_loss_body,
        grid=(_TC_BLOCKS,),
        in_specs=[
            pl.BlockSpec((Bb, _W), lambda i: (i, 0)),
            pl.BlockSpec((Bb, _PACK), lambda i: (i, 0)),
            pl.BlockSpec((Rb, 128), lambda i: (i, 0)),
            pl.BlockSpec((_PACK, _W), lambda i: (0, 0)),
            pl.BlockSpec((1, _D), lambda i: (0, 0)),
        ],
        out_shape=jax.ShapeDtypeStruct((1, 1), jnp.float32),
        out_specs=pl.BlockSpec(memory_space=pltpu.SMEM),
    )(gathered, oh, y2d, u4, u)


def kernel(item, matrix, user_embeddings, item_embeddings):
    B = item.shape[0]
    V = item_embeddings.shape[0]
    try:
        info = plsc.get_sparse_core_info()
        NC, NS = info.num_cores, info.num_subcores
    except Exception:
        NC, NS = 2, 16
    NW = NC * NS
    b_per_w = B // NW
    n_chunks = b_per_w // _CHUNK

    item = item.astype(jnp.int32)
    table4 = item_embeddings.reshape(V // _PACK, _W)
    item_q = (item // _PACK).reshape(NW, n_chunks, _CHUNK)
    gathered = _sc_gather_fn(B, NC, NS)(item_q, table4)

    u = user_embeddings.reshape(1, _D).astype(jnp.float32)
    u4 = jnp.zeros((_PACK, _W), jnp.float32)
    for s in range(_PACK):
        u4 = u4.at[s, s * _D:(s + 1) * _D].set(u[0])
    oh = (
        (item % _PACK)[:, None] == jnp.arange(_PACK, dtype=jnp.int32)[None, :]
    ).astype(jnp.float32)
    out = _tc_loss(gathered, oh, matrix.reshape(128, 128), u4, u)
    return out[0, 0]


# speedup vs baseline: 4.6449x; 4.6449x over previous
import functools

import jax
import jax.numpy as jnp
from jax import lax
from jax.experimental import pallas as pl
from jax.experimental.pallas import tpu as pltpu
from jax.experimental.pallas import tpu_sc as plsc

_LAM_U = 0.01
_D = 32
_CHUNK = 128
_MV_W = 8192


def _matvec_body(t_ref, u_ref, o_ref):
    x = t_ref[...]
    u = u_ref[...]
    o_ref[...] = jnp.sum(x * u, axis=0)


@functools.cache
def _matvec_fn(V: int):
    grid = (V + _MV_W - 1) // _MV_W
    return pl.pallas_call(
        _matvec_body,
        grid=(grid,),
        in_specs=[
            pl.BlockSpec((_D, _MV_W), lambda i: (0, i)),
            pl.BlockSpec((_D, 1), lambda i: (0, 0)),
        ],
        out_specs=pl.BlockSpec((_MV_W,), lambda i: (i,)),
        out_shape=jax.ShapeDtypeStruct((V,), jnp.float32),
    )


@functools.cache
def _sc_gather_fn(B: int, V: int, NC: int, NS: int):
    NW = NC * NS
    b_per_w = B // NW
    n_chunks = b_per_w // _CHUNK
    mesh = plsc.VectorSubcoreMesh(core_axis_name="c", subcore_axis_name="s")

    @functools.partial(
        pl.kernel,
        mesh=mesh,
        compiler_params=pltpu.CompilerParams(use_tc_tiling_on_sc=False),
        out_type=jax.ShapeDtypeStruct((B,), jnp.float32),
        scratch_types=[
            pltpu.VMEM((n_chunks, _CHUNK), jnp.int32),
            pltpu.VMEM((b_per_w,), jnp.float32),
            pltpu.SemaphoreType.DMA,
        ],
    )
    def sc_gather(item_hbm, logits_hbm, out_hbm, idx_v, g_v, sem):
        wid = lax.axis_index("s") * NC + lax.axis_index("c")
        base = wid * b_per_w
        pltpu.sync_copy(item_hbm.at[wid], idx_v)
        copies = []
        for j in range(n_chunks):
            copies.append(pltpu.async_copy(
                logits_hbm.at[idx_v.at[j]],
                g_v.at[pl.ds(j * _CHUNK, _CHUNK)],
                sem))
        for c in copies:
            c.wait()
        pltpu.sync_copy(g_v, out_hbm.at[pl.ds(base, b_per_w)])

    return sc_gather


def _tc_loss_body(x_ref, y_ref, u_ref, o_ref):
    x = x_ref[...]
    y = y_ref[...]
    bce = jnp.maximum(x, 0.0) - x * y + jnp.log1p(jnp.exp(-jnp.abs(x)))
    u = u_ref[...]
    o_ref[0, 0] = jnp.sum(bce) + _LAM_U * jnp.sqrt(jnp.sum(u * u))


def _tc_loss(logits2d, y2d, u):
    return pl.pallas_call(
        _tc_loss_body,
        out_shape=jax.ShapeDtypeStruct((1, 1), jnp.float32),
        out_specs=pl.BlockSpec(memory_space=pltpu.SMEM),
    )(logits2d, y2d, u)


def kernel(item, matrix, user_embeddings, item_embeddings):
    B = item.shape[0]
    V = item_embeddings.shape[0]
    try:
        info = plsc.get_sparse_core_info()
        NC, NS = info.num_cores, info.num_subcores
    except Exception:
        NC, NS = 2, 16
    NW = NC * NS
    b_per_w = B // NW
    n_chunks = b_per_w // _CHUNK

    tview = item_embeddings.T
    u_col = user_embeddings.reshape(_D, 1).astype(jnp.float32)
    logits_all = _matvec_fn(V)(tview, u_col)

    item_r = item.astype(jnp.int32).reshape(NW, n_chunks, _CHUNK)
    logits = _sc_gather_fn(B, V, NC, NS)(item_r, logits_all)

    u = user_embeddings.reshape(1, _D).astype(jnp.float32)
    out = _tc_loss(logits.reshape(128, 128), matrix.reshape(128, 128), u)
    return out[0, 0]


# speedup vs baseline: 6.1831x; 1.3312x over previous
import functools

import jax
import jax.numpy as jnp
from jax import lax
from jax.experimental import pallas as pl
from jax.experimental.pallas import tpu as pltpu
from jax.experimental.pallas import tpu_sc as plsc

_LAM_U = 0.01
_D = 32
_CHUNK = 128
_MV_W = 16384


def _matvec_body(t_ref, u_ref, o_ref):
    x = t_ref[...]
    u = u_ref[...]
    o_ref[...] = jnp.sum(x * u, axis=0)


@functools.cache
def _matvec_fn(V: int):
    grid = (V + _MV_W - 1) // _MV_W
    return pl.pallas_call(
        _matvec_body,
        grid=(grid,),
        in_specs=[
            pl.BlockSpec((_D, _MV_W), lambda i: (0, i)),
            pl.BlockSpec((_D, 1), lambda i: (0, 0)),
        ],
        out_specs=pl.BlockSpec((_MV_W,), lambda i: (i,)),
        out_shape=jax.ShapeDtypeStruct((V,), jnp.float32),
    )


@functools.cache
def _sc_gather_fn(B: int, V: int, NC: int, NS: int):
    NW = NC * NS
    b_per_w = B // NW
    n_chunks = b_per_w // _CHUNK
    mesh = plsc.VectorSubcoreMesh(core_axis_name="c", subcore_axis_name="s")

    @functools.partial(
        pl.kernel,
        mesh=mesh,
        compiler_params=pltpu.CompilerParams(use_tc_tiling_on_sc=False),
        out_type=jax.ShapeDtypeStruct((B,), jnp.float32),
        scratch_types=[
            pltpu.VMEM((n_chunks, _CHUNK), jnp.int32),
            pltpu.VMEM((b_per_w,), jnp.float32),
            pltpu.SemaphoreType.DMA,
        ],
    )
    def sc_gather(item_hbm, logits_hbm, out_hbm, idx_v, g_v, sem):
        wid = lax.axis_index("s") * NC + lax.axis_index("c")
        base = wid * b_per_w
        pltpu.sync_copy(item_hbm.at[wid], idx_v)
        copies = []
        for j in range(n_chunks):
            copies.append(pltpu.async_copy(
                logits_hbm.at[idx_v.at[j]],
                g_v.at[pl.ds(j * _CHUNK, _CHUNK)],
                sem))
        for c in copies:
            c.wait()
        pltpu.sync_copy(g_v, out_hbm.at[pl.ds(base, b_per_w)])

    return sc_gather


def _tc_loss_body(x_ref, y_ref, u_ref, o_ref):
    x = x_ref[...]
    y = y_ref[...]
    bce = jnp.maximum(x, 0.0) - x * y + jnp.log1p(jnp.exp(-jnp.abs(x)))
    u = u_ref[...]
    o_ref[0, 0] = jnp.sum(bce) + _LAM_U * jnp.sqrt(jnp.sum(u * u))


def _tc_loss(logits2d, y2d, u):
    return pl.pallas_call(
        _tc_loss_body,
        out_shape=jax.ShapeDtypeStruct((1, 1), jnp.float32),
        out_specs=pl.BlockSpec(memory_space=pltpu.SMEM),
    )(logits2d, y2d, u)


def kernel(item, matrix, user_embeddings, item_embeddings):
    B = item.shape[0]
    V = item_embeddings.shape[0]
    try:
        info = plsc.get_sparse_core_info()
        NC, NS = info.num_cores, info.num_subcores
    except Exception:
        NC, NS = 2, 16
    NW = NC * NS
    b_per_w = B // NW
    n_chunks = b_per_w // _CHUNK

    tview = item_embeddings.T
    u_col = user_embeddings.reshape(_D, 1).astype(jnp.float32)
    logits_all = _matvec_fn(V)(tview, u_col)

    item_r = item.astype(jnp.int32).reshape(NW, n_chunks, _CHUNK)
    logits = _sc_gather_fn(B, V, NC, NS)(item_r, logits_all)

    u = user_embeddings.reshape(1, _D).astype(jnp.float32)
    out = _tc_loss(logits.reshape(128, 128), matrix.reshape(128, 128), u)
    return out[0, 0]


# speedup vs baseline: 8.4303x; 1.3634x over previous
import functools

import jax
import jax.numpy as jnp
from jax import lax
from jax.experimental import pallas as pl
from jax.experimental.pallas import tpu as pltpu
from jax.experimental.pallas import tpu_sc as plsc

_LAM_U = 0.01
_D = 32
_CHUNK = 128
_MV_W = 65536


def _matvec_body(t_ref, u_ref, o_ref):
    x = t_ref[...]
    u = u_ref[...]
    o_ref[...] = jnp.sum(x * u, axis=0)


@functools.cache
def _matvec_fn(V: int):
    grid = (V + _MV_W - 1) // _MV_W
    return pl.pallas_call(
        _matvec_body,
        grid=(grid,),
        in_specs=[
            pl.BlockSpec((_D, _MV_W), lambda i: (0, i)),
            pl.BlockSpec((_D, 1), lambda i: (0, 0)),
        ],
        out_specs=pl.BlockSpec((_MV_W,), lambda i: (i,)),
        out_shape=jax.ShapeDtypeStruct((V,), jnp.float32),
    )


@functools.cache
def _sc_gather_fn(B: int, V: int, NC: int, NS: int):
    NW = NC * NS
    b_per_w = B // NW
    n_chunks = b_per_w // _CHUNK
    mesh = plsc.VectorSubcoreMesh(core_axis_name="c", subcore_axis_name="s")

    @functools.partial(
        pl.kernel,
        mesh=mesh,
        compiler_params=pltpu.CompilerParams(use_tc_tiling_on_sc=False),
        out_type=jax.ShapeDtypeStruct((B,), jnp.float32),
        scratch_types=[
            pltpu.VMEM((n_chunks, _CHUNK), jnp.int32),
            pltpu.VMEM((b_per_w,), jnp.float32),
            pltpu.SemaphoreType.DMA,
        ],
    )
    def sc_gather(item_hbm, logits_hbm, out_hbm, idx_v, g_v, sem):
        wid = lax.axis_index("s") * NC + lax.axis_index("c")
        base = wid * b_per_w
        pltpu.sync_copy(item_hbm.at[wid], idx_v)
        copies = []
        for j in range(n_chunks):
            copies.append(pltpu.async_copy(
                logits_hbm.at[idx_v.at[j]],
                g_v.at[pl.ds(j * _CHUNK, _CHUNK)],
                sem))
        for c in copies:
            c.wait()
        pltpu.sync_copy(g_v, out_hbm.at[pl.ds(base, b_per_w)])

    return sc_gather


def _tc_loss_body(x_ref, y_ref, u_ref, o_ref):
    x = x_ref[...]
    y = y_ref[...]
    bce = jnp.maximum(x, 0.0) - x * y + jnp.log1p(jnp.exp(-jnp.abs(x)))
    u = u_ref[...]
    o_ref[0, 0] = jnp.sum(bce) + _LAM_U * jnp.sqrt(jnp.sum(u * u))


def _tc_loss(logits2d, y2d, u):
    return pl.pallas_call(
        _tc_loss_body,
        out_shape=jax.ShapeDtypeStruct((1, 1), jnp.float32),
        out_specs=pl.BlockSpec(memory_space=pltpu.SMEM),
    )(logits2d, y2d, u)


def kernel(item, matrix, user_embeddings, item_embeddings):
    B = item.shape[0]
    V = item_embeddings.shape[0]
    try:
        info = plsc.get_sparse_core_info()
        NC, NS = info.num_cores, info.num_subcores
    except Exception:
        NC, NS = 2, 16
    NW = NC * NS
    b_per_w = B // NW
    n_chunks = b_per_w // _CHUNK

    tview = item_embeddings.T
    u_col = user_embeddings.reshape(_D, 1).astype(jnp.float32)
    logits_all = _matvec_fn(V)(tview, u_col)

    item_r = item.astype(jnp.int32).reshape(NW, n_chunks, _CHUNK)
    logits = _sc_gather_fn(B, V, NC, NS)(item_r, logits_all)

    u = user_embeddings.reshape(1, _D).astype(jnp.float32)
    out = _tc_loss(logits.reshape(128, 128), matrix.reshape(128, 128), u)
    return out[0, 0]
